# Initial kernel scaffold; baseline (speedup 1.0000x reference)
import jax
import jax.numpy as jnp
from jax.experimental import pallas as pl


def kernel(x, support, W1, W2, Wc, bc):
    raise NotImplementedError("write your pallas kernel here")


# speedup vs baseline: 1.8331x; 1.8331x over previous
import jax
import jax.numpy as jnp
from jax.experimental import pallas as pl
from jax.experimental.pallas import tpu as pltpu

_N = 2048
_D_IN = 512
_D_H = 2048
_D_OUT = 16
_RB = 256
_NB = _N // _RB


def _gcn_kernel(x_ref, sup_ref, w1_ref, w2_ref, wc_ref, bc_ref,
                out_ref, b_ref, acc_ref):
    p = pl.program_id(0)
    i = pl.program_id(1)

    @pl.when(p == 0)
    def _phase_a():
        a = jnp.dot(sup_ref[...], x_ref[...],
                    preferred_element_type=jnp.float32)
        h1 = jnp.maximum(
            jnp.dot(a, w1_ref[...], preferred_element_type=jnp.float32), 0.0)
        b = jnp.dot(h1, w2_ref[...], preferred_element_type=jnp.float32)
        b_ref[pl.ds(i * _RB, _RB), :] = b.astype(jnp.bfloat16)

    @pl.when(p == 1)
    def _phase_b():
        @pl.when(i == 0)
        def _init():
            acc_ref[...] = jnp.zeros_like(acc_ref)

        sup_bf = sup_ref[...].astype(jnp.bfloat16)
        h2 = jnp.maximum(
            jnp.dot(sup_bf, b_ref[...], preferred_element_type=jnp.float32),
            0.0)
        rs = jnp.sum(h2, axis=1, keepdims=True)
        acc_ref[...] += jnp.sum(rs * wc_ref[...], axis=0, keepdims=True)

        @pl.when(i == _NB - 1)
        def _final():
            logits = acc_ref[...] * (1.0 / _D_H) + bc_ref[...]
            mx = jnp.max(logits, axis=1, keepdims=True)
            e = jnp.exp(logits - mx)
            out_ref[...] = e / jnp.sum(e, axis=1, keepdims=True)


def kernel(x, support, W1, W2, Wc, bc):
    bc2 = bc.reshape(1, _D_OUT)
    return pl.pallas_call(
        _gcn_kernel,
        grid=(2, _NB),
        in_specs=[
            pl.BlockSpec((_N, _D_IN), lambda p, i: (0, 0)),
            pl.BlockSpec((_RB, _N), lambda p, i: (i, 0)),
            pl.BlockSpec((_D_IN, _D_H), lambda p, i: (0, 0)),
            pl.BlockSpec((_D_H, _D_H), lambda p, i: (0, 0)),
            pl.BlockSpec((_RB, _D_OUT), lambda p, i: (i, 0)),
            pl.BlockSpec((1, _D_OUT), lambda p, i: (0, 0)),
        ],
        out_specs=pl.BlockSpec((1, _D_OUT), lambda p, i: (0, 0)),
        out_shape=jax.ShapeDtypeStruct((1, _D_OUT), jnp.float32),
        scratch_shapes=[
            pltpu.VMEM((_N, _D_H), jnp.bfloat16),
            pltpu.VMEM((1, _D_OUT), jnp.float32),
        ],
        compiler_params=pltpu.CompilerParams(
            vmem_limit_bytes=60 * 1024 * 1024),
    )(x, support, W1, W2, Wc, bc2)


# speedup vs baseline: 1.9313x; 1.0536x over previous
import jax
import jax.numpy as jnp
from jax.experimental import pallas as pl
from jax.experimental.pallas import tpu as pltpu

_N = 2048
_D_IN = 512
_D_H = 2048
_D_OUT = 16
_RB = 512
_NB = _N // _RB


def _gcn_kernel(x_ref, sup_ref, w1_ref, w2_ref, wc_ref, bc_ref,
                out_ref, b_ref, acc_ref):
    p = pl.program_id(0)
    i = pl.program_id(1)

    @pl.when(p == 0)
    def _phase_a():
        a = jnp.dot(sup_ref[...], x_ref[...],
                    preferred_element_type=jnp.float32)
        h1 = jnp.maximum(
            jnp.dot(a, w1_ref[...], preferred_element_type=jnp.float32), 0.0)
        b = jnp.dot(h1, w2_ref[...], preferred_element_type=jnp.float32)
        b_ref[pl.ds(i * _RB, _RB), :] = b.astype(jnp.bfloat16)

    @pl.when(p == 1)
    def _phase_b():
        @pl.when(i == 0)
        def _init():
            acc_ref[...] = jnp.zeros_like(acc_ref)

        sup_bf = sup_ref[...].astype(jnp.bfloat16)
        h2 = jnp.maximum(
            jnp.dot(sup_bf, b_ref[...], preferred_element_type=jnp.float32),
            0.0)
        rs = jnp.sum(h2, axis=1, keepdims=True)
        acc_ref[...] += jnp.sum(rs * wc_ref[...], axis=0, keepdims=True)

        @pl.when(i == _NB - 1)
        def _final():
            logits = acc_ref[...] * (1.0 / _D_H) + bc_ref[...]
            mx = jnp.max(logits, axis=1, keepdims=True)
            e = jnp.exp(logits - mx)
            out_ref[...] = e / jnp.sum(e, axis=1, keepdims=True)


def kernel(x, support, W1, W2, Wc, bc):
    bc2 = bc.reshape(1, _D_OUT)
    return pl.pallas_call(
        _gcn_kernel,
        grid=(2, _NB),
        in_specs=[
            pl.BlockSpec((_N, _D_IN), lambda p, i: (0, 0)),
            pl.BlockSpec((_RB, _N), lambda p, i: (i, 0)),
            pl.BlockSpec((_D_IN, _D_H), lambda p, i: (0, 0)),
            pl.BlockSpec((_D_H, _D_H), lambda p, i: (0, 0)),
            pl.BlockSpec((_RB, _D_OUT), lambda p, i: (i, 0)),
            pl.BlockSpec((1, _D_OUT), lambda p, i: (0, 0)),
        ],
        out_specs=pl.BlockSpec((1, _D_OUT), lambda p, i: (0, 0)),
        out_shape=jax.ShapeDtypeStruct((1, _D_OUT), jnp.float32),
        scratch_shapes=[
            pltpu.VMEM((_N, _D_H), jnp.bfloat16),
            pltpu.VMEM((1, _D_OUT), jnp.float32),
        ],
        compiler_params=pltpu.CompilerParams(
            vmem_limit_bytes=60 * 1024 * 1024),
    )(x, support, W1, W2, Wc, bc2)


# speedup vs baseline: 1.9949x; 1.0329x over previous
import jax
import jax.numpy as jnp
from jax.experimental import pallas as pl
from jax.experimental.pallas import tpu as pltpu

_N = 2048
_D_IN = 512
_D_H = 2048
_D_OUT = 16
_RB = 512
_NB = _N // _RB
_CB = _D_H // _NB


def _gcn_kernel(x_ref, sup_ref, w1_ref, w2_ref, wc_ref, bc_ref,
                out_ref, h1_ref, b_ref, acc_ref):
    p = pl.program_id(0)
    i = pl.program_id(1)

    @pl.when(p == 0)
    def _phase_h1():
        a = jnp.dot(sup_ref[...], x_ref[...],
                    preferred_element_type=jnp.float32)
        h1 = jnp.maximum(
            jnp.dot(a, w1_ref[...], preferred_element_type=jnp.float32), 0.0)
        h1_ref[pl.ds(i * _RB, _RB), :] = h1.astype(jnp.bfloat16)

    @pl.when(p == 1)
    def _phase_b():
        w2_bf = w2_ref[...].astype(jnp.bfloat16)
        b = jnp.dot(h1_ref[...], w2_bf,
                    preferred_element_type=jnp.float32)
        b_ref[:, pl.ds(i * _CB, _CB)] = b.astype(jnp.bfloat16)

    @pl.when(p == 2)
    def _phase_h2():
        @pl.when(i == 0)
        def _init():
            acc_ref[...] = jnp.zeros_like(acc_ref)

        sup_bf = sup_ref[...].astype(jnp.bfloat16)
        h2 = jnp.maximum(
            jnp.dot(sup_bf, b_ref[...], preferred_element_type=jnp.float32),
            0.0)
        rs = jnp.sum(h2, axis=1, keepdims=True)
        acc_ref[...] += jnp.sum(rs * wc_ref[...], axis=0, keepdims=True)

        @pl.when(i == _NB - 1)
        def _final():
            logits = acc_ref[...] * (1.0 / _D_H) + bc_ref[...]
            mx = jnp.max(logits, axis=1, keepdims=True)
            e = jnp.exp(logits - mx)
            out_ref[...] = e / jnp.sum(e, axis=1, keepdims=True)


def kernel(x, support, W1, W2, Wc, bc):
    bc2 = bc.reshape(1, _D_OUT)
    last = _NB - 1
    return pl.pallas_call(
        _gcn_kernel,
        grid=(3, _NB),
        in_specs=[
            pl.BlockSpec((_N, _D_IN), lambda p, i: (0, 0)),
            pl.BlockSpec((_RB, _N),
                         lambda p, i: (jnp.where(p == 1, last, i), 0)),
            pl.BlockSpec((_D_IN, _D_H), lambda p, i: (0, 0)),
            pl.BlockSpec((_D_H, _CB),
                         lambda p, i: (0, jnp.where(p == 1, i, 0))),
            pl.BlockSpec((_RB, _D_OUT),
                         lambda p, i: (jnp.where(p == 2, i, 0), 0)),
            pl.BlockSpec((1, _D_OUT), lambda p, i: (0, 0)),
        ],
        out_specs=pl.BlockSpec((1, _D_OUT), lambda p, i: (0, 0)),
        out_shape=jax.ShapeDtypeStruct((1, _D_OUT), jnp.float32),
        scratch_shapes=[
            pltpu.VMEM((_N, _D_H), jnp.bfloat16),
            pltpu.VMEM((_N, _D_H), jnp.bfloat16),
            pltpu.VMEM((1, _D_OUT), jnp.float32),
        ],
        compiler_params=pltpu.CompilerParams(
            vmem_limit_bytes=60 * 1024 * 1024),
    )(x, support, W1, W2, Wc, bc2)
